# baseline (speedup 1.0000x reference)
import functools

import numpy as np
import jax
import jax.numpy as jnp
from jax.experimental import pallas as pl
from jax.experimental.pallas import tpu as pltpu

B = 16
EMBED_DIMS = (32, 64, 128, 160)
OUT_HW = (256, 256)
_VMEM_LIMIT = 48 * 1024 * 1024


def _ln_f32(x, g, b, eps=1e-5):
    mu = jnp.mean(x, axis=-1, keepdims=True)
    xc = x - mu
    var = jnp.mean(xc * xc, axis=-1, keepdims=True)
    return xc * jax.lax.rsqrt(var + eps) * g + b


def _encoder_math(xs, ws, eb, g1, b1, g2, b2, w1, bb1, w2, bb2):
    tok = jnp.dot(xs[0], ws[0], preferred_element_type=jnp.float32)
    for xv, wv in zip(xs[1:], ws[1:]):
        tok = tok + jnp.dot(xv, wv, preferred_element_type=jnp.float32)
    tok = tok + eb
    tok = _ln_f32(tok, g1, b1)
    h = _ln_f32(tok, g2, b2)
    h = jnp.dot(h.astype(jnp.bfloat16), w1, preferred_element_type=jnp.float32) + bb1
    h = jax.nn.gelu(h, approximate=True)
    return (jnp.dot(h.astype(jnp.bfloat16), w2,
                    preferred_element_type=jnp.float32) + bb2 + tok)


def _regroup(out, hp, wp, c):
    o = out.reshape(hp // 2, 2, wp // 2, 2, c)
    parts = [o[:, dy, :, dx, :] for dy in (0, 1) for dx in (0, 1)]
    p = jnp.concatenate(parts, axis=-1)
    return p.reshape((hp * wp) // 4, 4 * c).astype(jnp.bfloat16)


def _stage0_body(pa_ref, pb_ref, pc_ref, wa_ref, wb_ref, wc_ref, eb_ref,
                 g1_ref, b1_ref, g2_ref, b2_ref, w1_ref, bb1_ref, w2_ref,
                 bb2_ref, o_ref, p_ref):
    out = _encoder_math(
        [pa_ref[...], pb_ref[...], pc_ref[...]],
        [wa_ref[...], wb_ref[...], wc_ref[...]],
        eb_ref[...], g1_ref[...], b1_ref[...], g2_ref[...], b2_ref[...],
        w1_ref[...], bb1_ref[...], w2_ref[...], bb2_ref[...])
    o_ref[...] = out.T.reshape(1, 32, 64, 64)
    p_ref[...] = _regroup(out, 64, 64, 32)


def _stage_body(p_in_ref, w_ref, eb_ref, g1_ref, b1_ref, g2_ref, b2_ref,
                w1_ref, bb1_ref, w2_ref, bb2_ref, o_ref, p_ref, *, hp, wp, c):
    out = _encoder_math(
        [p_in_ref[...]], [w_ref[...]],
        eb_ref[...], g1_ref[...], b1_ref[...], g2_ref[...], b2_ref[...],
        w1_ref[...], bb1_ref[...], w2_ref[...], bb2_ref[...])
    o_ref[...] = out.T.reshape(1, c, hp, wp)
    p_ref[...] = _regroup(out, hp, wp, c)


def _stage3_body(p_in_ref, w_ref, eb_ref, g1_ref, b1_ref, g2_ref, b2_ref,
                 w1_ref, bb1_ref, w2_ref, bb2_ref, o_ref, t_ref):
    out = _encoder_math(
        [p_in_ref[...]], [w_ref[...]],
        eb_ref[...], g1_ref[...], b1_ref[...], g2_ref[...], b2_ref[...],
        w1_ref[...], bb1_ref[...], w2_ref[...], bb2_ref[...])
    t = out.reshape(8, 64, 160)
    o_ref[...] = jnp.transpose(t, (0, 2, 1)).reshape(8, 160, 8, 8)
    t_ref[...] = out


def _full(shape):
    return pl.BlockSpec(shape, lambda *_, _s=shape: tuple(0 for _ in _s))


def _stage_params(st, c):
    h = 4 * c
    return (st["embed_b"].reshape(1, c),
            st["ln1_g"].reshape(1, c), st["ln1_b"].reshape(1, c),
            st["ln2_g"].reshape(1, c), st["ln2_b"].reshape(1, c),
            st["mlp_w1"], st["mlp_b1"].reshape(1, h),
            st["mlp_w2"], st["mlp_b2"].reshape(1, c))


def _stage_param_specs(kin, c):
    h = 4 * c
    return [_full((1, c)), _full((1, c)), _full((1, c)), _full((1, c)),
            _full((1, c)), _full((c, h)), _full((1, h)), _full((h, c)),
            _full((1, c))]


def _compiler_params():
    return pltpu.CompilerParams(
        dimension_semantics=("parallel", "arbitrary"),
        vmem_limit_bytes=_VMEM_LIMIT)


def _bilinear_matrix_np(out_size, in_size):
    dst = np.arange(out_size, dtype=np.float32)
    scale = in_size / out_size
    src = np.clip((dst + 0.5) * scale - 0.5, 0.0, in_size - 1)
    i0 = np.floor(src).astype(np.int32)
    i1 = np.minimum(i0 + 1, in_size - 1)
    lam = (src - i0).astype(np.float32)
    a = np.zeros((out_size, in_size), np.float32)
    rows = np.arange(out_size)
    a[rows, i0] += 1.0 - lam
    a[rows, i1] += lam
    return a


@functools.lru_cache(maxsize=None)
def _bilinear_factors(h, w, oh, ow):
    a = jnp.asarray(_bilinear_matrix_np(oh, h))
    bt = jnp.asarray(_bilinear_matrix_np(ow, w).T)
    return a, bt


def _head_body(f_ref, hw_ref, hb_ref, a_ref, bt_ref, o_ref):
    feat = f_ref[0]
    hv = jnp.sum(feat * hw_ref[...], axis=-1, keepdims=True) + hb_ref[0, 0]
    img = hv.reshape(8, 8)
    tmp = jnp.dot(img, bt_ref[...], preferred_element_type=jnp.float32)
    o_ref[0] = jnp.dot(a_ref[...], tmp, preferred_element_type=jnp.float32)


def _s0_perm(g):
    idx = np.empty((48,), np.int32)
    k = 0
    for ch in range(3):
        for dy in range(4):
            for dx in range(4):
                idx[k] = (dy * 4 + dx) * 9 + 3 * g + ch
                k += 1
    return idx


@functools.lru_cache(maxsize=None)
def _patch_eye():
    eye = np.zeros((48, 3, 4, 4), np.float32)
    for ch in range(3):
        for dy in range(4):
            for dx in range(4):
                eye[ch * 16 + dy * 4 + dx, ch, dy, dx] = 1.0
    return jnp.asarray(eye, jnp.bfloat16)


def _patchify0(img):
    out = jax.lax.conv_general_dilated(
        img.astype(jnp.bfloat16), _patch_eye(),
        window_strides=(4, 4), padding='VALID',
        dimension_numbers=('NCHW', 'OIHW', 'NHWC'),
        preferred_element_type=jnp.bfloat16)
    return out.reshape(B * 4096, 48)


def kernel(x, y, xstack0, xstack1,
           s0_embed_w, s0_embed_b, s0_ln1_g, s0_ln1_b, s0_ln2_g, s0_ln2_b,
           s0_mlp_w1, s0_mlp_b1, s0_mlp_w2, s0_mlp_b2,
           s1_embed_w, s1_embed_b, s1_ln1_g, s1_ln1_b, s1_ln2_g, s1_ln2_b,
           s1_mlp_w1, s1_mlp_b1, s1_mlp_w2, s1_mlp_b2,
           s2_embed_w, s2_embed_b, s2_ln1_g, s2_ln1_b, s2_ln2_g, s2_ln2_b,
           s2_mlp_w1, s2_mlp_b1, s2_mlp_w2, s2_mlp_b2,
           s3_embed_w, s3_embed_b, s3_ln1_g, s3_ln1_b, s3_ln2_g, s3_ln2_b,
           s3_mlp_w1, s3_mlp_b1, s3_mlp_w2, s3_mlp_b2,
           head_w, head_b):
    stages = [
        dict(embed_w=s0_embed_w, embed_b=s0_embed_b, ln1_g=s0_ln1_g,
             ln1_b=s0_ln1_b, ln2_g=s0_ln2_g, ln2_b=s0_ln2_b,
             mlp_w1=s0_mlp_w1, mlp_b1=s0_mlp_b1, mlp_w2=s0_mlp_w2,
             mlp_b2=s0_mlp_b2),
        dict(embed_w=s1_embed_w, embed_b=s1_embed_b, ln1_g=s1_ln1_g,
             ln1_b=s1_ln1_b, ln2_g=s1_ln2_g, ln2_b=s1_ln2_b,
             mlp_w1=s1_mlp_w1, mlp_b1=s1_mlp_b1, mlp_w2=s1_mlp_w2,
             mlp_b2=s1_mlp_b2),
        dict(embed_w=s2_embed_w, embed_b=s2_embed_b, ln1_g=s2_ln1_g,
             ln1_b=s2_ln1_b, ln2_g=s2_ln2_g, ln2_b=s2_ln2_b,
             mlp_w1=s2_mlp_w1, mlp_b1=s2_mlp_b1, mlp_w2=s2_mlp_w2,
             mlp_b2=s2_mlp_b2),
        dict(embed_w=s3_embed_w, embed_b=s3_embed_b, ln1_g=s3_ln1_g,
             ln1_b=s3_ln1_b, ln2_g=s3_ln2_g, ln2_b=s3_ln2_b,
             mlp_w1=s3_mlp_w1, mlp_b1=s3_mlp_b1, mlp_w2=s3_mlp_w2,
             mlp_b2=s3_mlp_b2),
    ]

    pa = _patchify0(xstack0)
    pb = _patchify0(xstack1)
    pc = _patchify0(y)
    wa = s0_embed_w[jnp.asarray(_s0_perm(0))]
    wb = s0_embed_w[jnp.asarray(_s0_perm(1))]
    wc = s0_embed_w[jnp.asarray(_s0_perm(2))]

    st = stages[0]
    out1, p1 = pl.pallas_call(
        _stage0_body,
        out_shape=(jax.ShapeDtypeStruct((B, 32, 64, 64), jnp.float32),
                   jax.ShapeDtypeStruct((B * 1024, 128), jnp.bfloat16)),
        grid=(2, 8),
        in_specs=[
            pl.BlockSpec((4096, 48), lambda c, j: (c * 8 + j, 0)),
            pl.BlockSpec((4096, 48), lambda c, j: (c * 8 + j, 0)),
            pl.BlockSpec((4096, 48), lambda c, j: (c * 8 + j, 0)),
            _full((48, 32)), _full((48, 32)), _full((48, 32)),
        ] + _stage_param_specs(144, 32),
        out_specs=(pl.BlockSpec((1, 32, 64, 64), lambda c, j: (c * 8 + j, 0, 0, 0)),
                   pl.BlockSpec((1024, 128), lambda c, j: (c * 8 + j, 0))),
        compiler_params=_compiler_params(),
    )(pa, pb, pc, wa, wb, wc, *_stage_params(st, 32))

    st = stages[1]
    out2, p2 = pl.pallas_call(
        functools.partial(_stage_body, hp=32, wp=32, c=64),
        out_shape=(jax.ShapeDtypeStruct((B, 64, 32, 32), jnp.float32),
                   jax.ShapeDtypeStruct((B * 256, 256), jnp.bfloat16)),
        grid=(2, 8),
        in_specs=[pl.BlockSpec((1024, 128), lambda c, j: (c * 8 + j, 0)),
                  _full((128, 64))] + _stage_param_specs(128, 64),
        out_specs=(pl.BlockSpec((1, 64, 32, 32), lambda c, j: (c * 8 + j, 0, 0, 0)),
                   pl.BlockSpec((256, 256), lambda c, j: (c * 8 + j, 0))),
        compiler_params=_compiler_params(),
    )(p1, st["embed_w"], *_stage_params(st, 64))

    st = stages[2]
    out3, p3 = pl.pallas_call(
        functools.partial(_stage_body, hp=16, wp=16, c=128),
        out_shape=(jax.ShapeDtypeStruct((B, 128, 16, 16), jnp.float32),
                   jax.ShapeDtypeStruct((B * 64, 512), jnp.bfloat16)),
        grid=(2, 8),
        in_specs=[pl.BlockSpec((256, 256), lambda c, j: (c * 8 + j, 0)),
                  _full((256, 128))] + _stage_param_specs(256, 128),
        out_specs=(pl.BlockSpec((1, 128, 16, 16), lambda c, j: (c * 8 + j, 0, 0, 0)),
                   pl.BlockSpec((64, 512), lambda c, j: (c * 8 + j, 0))),
        compiler_params=_compiler_params(),
    )(p2, st["embed_w"], *_stage_params(st, 128))

    st = stages[3]
    out4, tok4 = pl.pallas_call(
        _stage3_body,
        out_shape=(jax.ShapeDtypeStruct((B, 160, 8, 8), jnp.float32),
                   jax.ShapeDtypeStruct((B * 64, 160), jnp.float32)),
        grid=(2, 1),
        in_specs=[pl.BlockSpec((512, 512), lambda c, j: (c, 0)),
                  _full((512, 160))] + _stage_param_specs(512, 160),
        out_specs=(pl.BlockSpec((8, 160, 8, 8), lambda c, j: (c, 0, 0, 0)),
                   pl.BlockSpec((512, 160), lambda c, j: (c, 0))),
        compiler_params=_compiler_params(),
    )(p3, st["embed_w"], *_stage_params(st, 160))

    oh, ow = OUT_HW
    a_mat, bt_mat = _bilinear_factors(8, 8, oh, ow)
    feat = tok4.reshape(B, 64, 160)
    rgb = pl.pallas_call(
        _head_body,
        out_shape=jax.ShapeDtypeStruct((B, oh, ow), jnp.float32),
        grid=(2, 8),
        in_specs=[
            pl.BlockSpec((1, 64, 160), lambda c, j: (c * 8 + j, 0, 0)),
            _full((1, 160)), _full((1, 1)),
            _full((oh, 8)), _full((8, ow)),
        ],
        out_specs=pl.BlockSpec((1, oh, ow), lambda c, j: (c * 8 + j, 0, 0)),
        compiler_params=_compiler_params(),
    )(feat, head_w.reshape(1, 160), head_b.reshape(1, 1), a_mat, bt_mat)

    return rgb.reshape(B, 1, oh, ow), out1, out2, out3, out4


# speedup vs baseline: 17.1675x; 17.1675x over previous
import functools

import numpy as np
import jax
import jax.numpy as jnp
from jax.experimental import pallas as pl
from jax.experimental.pallas import tpu as pltpu

B = 16
EMBED_DIMS = (32, 64, 128, 160)
OUT_HW = (256, 256)
_VMEM_LIMIT = 48 * 1024 * 1024


def _ln_f32(x, g, b, eps=1e-5):
    mu = jnp.mean(x, axis=-1, keepdims=True)
    xc = x - mu
    var = jnp.mean(xc * xc, axis=-1, keepdims=True)
    return xc * jax.lax.rsqrt(var + eps) * g + b


def _encoder_math(xs, ws, eb, g1, b1, g2, b2, w1, bb1, w2, bb2):
    tok = jnp.dot(xs[0], ws[0], preferred_element_type=jnp.float32)
    for xv, wv in zip(xs[1:], ws[1:]):
        tok = tok + jnp.dot(xv, wv, preferred_element_type=jnp.float32)
    tok = tok + eb
    tok = _ln_f32(tok, g1, b1)
    h = _ln_f32(tok, g2, b2)
    h = jnp.dot(h.astype(jnp.bfloat16), w1, preferred_element_type=jnp.float32) + bb1
    h = jax.nn.gelu(h, approximate=True)
    return (jnp.dot(h.astype(jnp.bfloat16), w2,
                    preferred_element_type=jnp.float32) + bb2 + tok)


def _regroup(out, hp, wp, c):
    o = out.reshape(hp // 2, 2, wp // 2, 2, c)
    parts = [o[:, dy, :, dx, :] for dy in (0, 1) for dx in (0, 1)]
    p = jnp.concatenate(parts, axis=-1)
    return p.reshape((hp * wp) // 4, 4 * c).astype(jnp.bfloat16)


def _stage0_body(pa_ref, pb_ref, pc_ref, wa_ref, wb_ref, wc_ref, eb_ref,
                 g1_ref, b1_ref, g2_ref, b2_ref, w1_ref, bb1_ref, w2_ref,
                 bb2_ref, o_ref, p_ref):
    out = _encoder_math(
        [pa_ref[...], pb_ref[...], pc_ref[...]],
        [wa_ref[...], wb_ref[...], wc_ref[...]],
        eb_ref[...], g1_ref[...], b1_ref[...], g2_ref[...], b2_ref[...],
        w1_ref[...], bb1_ref[...], w2_ref[...], bb2_ref[...])
    o_ref[...] = out.T.reshape(1, 32, 64, 64)
    p_ref[...] = _regroup(out, 64, 64, 32)


def _stage_body(p_in_ref, w_ref, eb_ref, g1_ref, b1_ref, g2_ref, b2_ref,
                w1_ref, bb1_ref, w2_ref, bb2_ref, o_ref, p_ref, *, hp, wp, c):
    out = _encoder_math(
        [p_in_ref[...]], [w_ref[...]],
        eb_ref[...], g1_ref[...], b1_ref[...], g2_ref[...], b2_ref[...],
        w1_ref[...], bb1_ref[...], w2_ref[...], bb2_ref[...])
    o_ref[...] = out.T.reshape(1, c, hp, wp)
    p_ref[...] = _regroup(out, hp, wp, c)


def _stage3_body(p_in_ref, w_ref, eb_ref, g1_ref, b1_ref, g2_ref, b2_ref,
                 w1_ref, bb1_ref, w2_ref, bb2_ref, o_ref, t_ref):
    out = _encoder_math(
        [p_in_ref[...]], [w_ref[...]],
        eb_ref[...], g1_ref[...], b1_ref[...], g2_ref[...], b2_ref[...],
        w1_ref[...], bb1_ref[...], w2_ref[...], bb2_ref[...])
    t = out.reshape(8, 64, 160)
    o_ref[...] = jnp.transpose(t, (0, 2, 1)).reshape(8, 160, 8, 8)
    t_ref[...] = out


def _full(shape):
    return pl.BlockSpec(shape, lambda *_, _s=shape: tuple(0 for _ in _s))


def _stage_params(st, c):
    h = 4 * c
    return (st["embed_b"].reshape(1, c),
            st["ln1_g"].reshape(1, c), st["ln1_b"].reshape(1, c),
            st["ln2_g"].reshape(1, c), st["ln2_b"].reshape(1, c),
            st["mlp_w1"], st["mlp_b1"].reshape(1, h),
            st["mlp_w2"], st["mlp_b2"].reshape(1, c))


def _stage_param_specs(kin, c):
    h = 4 * c
    return [_full((1, c)), _full((1, c)), _full((1, c)), _full((1, c)),
            _full((1, c)), _full((c, h)), _full((1, h)), _full((h, c)),
            _full((1, c))]


def _compiler_params():
    return pltpu.CompilerParams(
        dimension_semantics=("parallel", "arbitrary"),
        vmem_limit_bytes=_VMEM_LIMIT)


def _bilinear_matrix_np(out_size, in_size):
    dst = np.arange(out_size, dtype=np.float32)
    scale = in_size / out_size
    src = np.clip((dst + 0.5) * scale - 0.5, 0.0, in_size - 1)
    i0 = np.floor(src).astype(np.int32)
    i1 = np.minimum(i0 + 1, in_size - 1)
    lam = (src - i0).astype(np.float32)
    a = np.zeros((out_size, in_size), np.float32)
    rows = np.arange(out_size)
    a[rows, i0] += 1.0 - lam
    a[rows, i1] += lam
    return a


@functools.lru_cache(maxsize=None)
def _bilinear_factors(h, w, oh, ow):
    a = jnp.asarray(_bilinear_matrix_np(oh, h))
    bt = jnp.asarray(_bilinear_matrix_np(ow, w).T)
    return a, bt


def _head_body(f_ref, hw_ref, hb_ref, a_ref, bt_ref, o_ref):
    feat = f_ref[0]
    hv = jnp.sum(feat * hw_ref[...], axis=-1, keepdims=True) + hb_ref[0, 0]
    img = hv.reshape(8, 8)
    tmp = jnp.dot(img, bt_ref[...], preferred_element_type=jnp.float32)
    o_ref[0] = jnp.dot(a_ref[...], tmp, preferred_element_type=jnp.float32)


def _s0_perm(g):
    idx = np.empty((48,), np.int32)
    k = 0
    for ch in range(3):
        for dy in range(4):
            for dx in range(4):
                idx[k] = (dy * 4 + dx) * 9 + 3 * g + ch
                k += 1
    return idx


def _patchify0(img):
    t = img.astype(jnp.bfloat16).reshape(B, 3, 64, 4, 64, 4)
    t = jnp.transpose(t, (0, 2, 4, 1, 3, 5))
    return t.reshape(B * 4096, 48)


def kernel(x, y, xstack0, xstack1,
           s0_embed_w, s0_embed_b, s0_ln1_g, s0_ln1_b, s0_ln2_g, s0_ln2_b,
           s0_mlp_w1, s0_mlp_b1, s0_mlp_w2, s0_mlp_b2,
           s1_embed_w, s1_embed_b, s1_ln1_g, s1_ln1_b, s1_ln2_g, s1_ln2_b,
           s1_mlp_w1, s1_mlp_b1, s1_mlp_w2, s1_mlp_b2,
           s2_embed_w, s2_embed_b, s2_ln1_g, s2_ln1_b, s2_ln2_g, s2_ln2_b,
           s2_mlp_w1, s2_mlp_b1, s2_mlp_w2, s2_mlp_b2,
           s3_embed_w, s3_embed_b, s3_ln1_g, s3_ln1_b, s3_ln2_g, s3_ln2_b,
           s3_mlp_w1, s3_mlp_b1, s3_mlp_w2, s3_mlp_b2,
           head_w, head_b):
    stages = [
        dict(embed_w=s0_embed_w, embed_b=s0_embed_b, ln1_g=s0_ln1_g,
             ln1_b=s0_ln1_b, ln2_g=s0_ln2_g, ln2_b=s0_ln2_b,
             mlp_w1=s0_mlp_w1, mlp_b1=s0_mlp_b1, mlp_w2=s0_mlp_w2,
             mlp_b2=s0_mlp_b2),
        dict(embed_w=s1_embed_w, embed_b=s1_embed_b, ln1_g=s1_ln1_g,
             ln1_b=s1_ln1_b, ln2_g=s1_ln2_g, ln2_b=s1_ln2_b,
             mlp_w1=s1_mlp_w1, mlp_b1=s1_mlp_b1, mlp_w2=s1_mlp_w2,
             mlp_b2=s1_mlp_b2),
        dict(embed_w=s2_embed_w, embed_b=s2_embed_b, ln1_g=s2_ln1_g,
             ln1_b=s2_ln1_b, ln2_g=s2_ln2_g, ln2_b=s2_ln2_b,
             mlp_w1=s2_mlp_w1, mlp_b1=s2_mlp_b1, mlp_w2=s2_mlp_w2,
             mlp_b2=s2_mlp_b2),
        dict(embed_w=s3_embed_w, embed_b=s3_embed_b, ln1_g=s3_ln1_g,
             ln1_b=s3_ln1_b, ln2_g=s3_ln2_g, ln2_b=s3_ln2_b,
             mlp_w1=s3_mlp_w1, mlp_b1=s3_mlp_b1, mlp_w2=s3_mlp_w2,
             mlp_b2=s3_mlp_b2),
    ]

    pa = _patchify0(xstack0)
    pb = _patchify0(xstack1)
    pc = _patchify0(y)
    wa = s0_embed_w[jnp.asarray(_s0_perm(0))]
    wb = s0_embed_w[jnp.asarray(_s0_perm(1))]
    wc = s0_embed_w[jnp.asarray(_s0_perm(2))]

    st = stages[0]
    out1, p1 = pl.pallas_call(
        _stage0_body,
        out_shape=(jax.ShapeDtypeStruct((B, 32, 64, 64), jnp.float32),
                   jax.ShapeDtypeStruct((B * 1024, 128), jnp.bfloat16)),
        grid=(2, 8),
        in_specs=[
            pl.BlockSpec((4096, 48), lambda c, j: (c * 8 + j, 0)),
            pl.BlockSpec((4096, 48), lambda c, j: (c * 8 + j, 0)),
            pl.BlockSpec((4096, 48), lambda c, j: (c * 8 + j, 0)),
            _full((48, 32)), _full((48, 32)), _full((48, 32)),
        ] + _stage_param_specs(144, 32),
        out_specs=(pl.BlockSpec((1, 32, 64, 64), lambda c, j: (c * 8 + j, 0, 0, 0)),
                   pl.BlockSpec((1024, 128), lambda c, j: (c * 8 + j, 0))),
        compiler_params=_compiler_params(),
    )(pa, pb, pc, wa, wb, wc, *_stage_params(st, 32))

    st = stages[1]
    out2, p2 = pl.pallas_call(
        functools.partial(_stage_body, hp=32, wp=32, c=64),
        out_shape=(jax.ShapeDtypeStruct((B, 64, 32, 32), jnp.float32),
                   jax.ShapeDtypeStruct((B * 256, 256), jnp.bfloat16)),
        grid=(2, 8),
        in_specs=[pl.BlockSpec((1024, 128), lambda c, j: (c * 8 + j, 0)),
                  _full((128, 64))] + _stage_param_specs(128, 64),
        out_specs=(pl.BlockSpec((1, 64, 32, 32), lambda c, j: (c * 8 + j, 0, 0, 0)),
                   pl.BlockSpec((256, 256), lambda c, j: (c * 8 + j, 0))),
        compiler_params=_compiler_params(),
    )(p1, st["embed_w"], *_stage_params(st, 64))

    st = stages[2]
    out3, p3 = pl.pallas_call(
        functools.partial(_stage_body, hp=16, wp=16, c=128),
        out_shape=(jax.ShapeDtypeStruct((B, 128, 16, 16), jnp.float32),
                   jax.ShapeDtypeStruct((B * 64, 512), jnp.bfloat16)),
        grid=(2, 8),
        in_specs=[pl.BlockSpec((256, 256), lambda c, j: (c * 8 + j, 0)),
                  _full((256, 128))] + _stage_param_specs(256, 128),
        out_specs=(pl.BlockSpec((1, 128, 16, 16), lambda c, j: (c * 8 + j, 0, 0, 0)),
                   pl.BlockSpec((64, 512), lambda c, j: (c * 8 + j, 0))),
        compiler_params=_compiler_params(),
    )(p2, st["embed_w"], *_stage_params(st, 128))

    st = stages[3]
    out4, tok4 = pl.pallas_call(
        _stage3_body,
        out_shape=(jax.ShapeDtypeStruct((B, 160, 8, 8), jnp.float32),
                   jax.ShapeDtypeStruct((B * 64, 160), jnp.float32)),
        grid=(2, 1),
        in_specs=[pl.BlockSpec((512, 512), lambda c, j: (c, 0)),
                  _full((512, 160))] + _stage_param_specs(512, 160),
        out_specs=(pl.BlockSpec((8, 160, 8, 8), lambda c, j: (c, 0, 0, 0)),
                   pl.BlockSpec((512, 160), lambda c, j: (c, 0))),
        compiler_params=_compiler_params(),
    )(p3, st["embed_w"], *_stage_params(st, 160))

    oh, ow = OUT_HW
    a_mat, bt_mat = _bilinear_factors(8, 8, oh, ow)
    feat = tok4.reshape(B, 64, 160)
    rgb = pl.pallas_call(
        _head_body,
        out_shape=jax.ShapeDtypeStruct((B, oh, ow), jnp.float32),
        grid=(2, 8),
        in_specs=[
            pl.BlockSpec((1, 64, 160), lambda c, j: (c * 8 + j, 0, 0)),
            _full((1, 160)), _full((1, 1)),
            _full((oh, 8)), _full((8, ow)),
        ],
        out_specs=pl.BlockSpec((1, oh, ow), lambda c, j: (c * 8 + j, 0, 0)),
        compiler_params=_compiler_params(),
    )(feat, head_w.reshape(1, 160), head_b.reshape(1, 1), a_mat, bt_mat)

    return rgb.reshape(B, 1, oh, ow), out1, out2, out3, out4


# speedup vs baseline: 26.9502x; 1.5698x over previous
import functools

import numpy as np
import jax
import jax.numpy as jnp
from jax.experimental import pallas as pl
from jax.experimental.pallas import tpu as pltpu

B = 16
EMBED_DIMS = (32, 64, 128, 160)
OUT_HW = (256, 256)
_VMEM_LIMIT = 48 * 1024 * 1024


def _ln_f32(x, g, b, eps=1e-5):
    mu = jnp.mean(x, axis=-1, keepdims=True)
    xc = x - mu
    var = jnp.mean(xc * xc, axis=-1, keepdims=True)
    return xc * jax.lax.rsqrt(var + eps) * g + b


def _encoder_math(xs, ws, eb, g1, b1, g2, b2, w1, bb1, w2, bb2):
    tok = jnp.dot(xs[0], ws[0], preferred_element_type=jnp.float32)
    for xv, wv in zip(xs[1:], ws[1:]):
        tok = tok + jnp.dot(xv, wv, preferred_element_type=jnp.float32)
    tok = tok + eb
    tok = _ln_f32(tok, g1, b1)
    h = _ln_f32(tok, g2, b2)
    h = jnp.dot(h.astype(jnp.bfloat16), w1, preferred_element_type=jnp.float32) + bb1
    h = jax.nn.gelu(h, approximate=True)
    return (jnp.dot(h.astype(jnp.bfloat16), w2,
                    preferred_element_type=jnp.float32) + bb2 + tok)


def _regroup(out, hp, wp, c):
    o = out.reshape(hp // 2, 2, wp // 2, 2, c)
    parts = [o[:, dy, :, dx, :] for dy in (0, 1) for dx in (0, 1)]
    p = jnp.concatenate(parts, axis=-1)
    return p.reshape((hp * wp) // 4, 4 * c).astype(jnp.bfloat16)


def _ln_ct(x, g, b, eps=1e-5):
    mu = jnp.mean(x, axis=0, keepdims=True)
    xc = x - mu
    var = jnp.mean(xc * xc, axis=0, keepdims=True)
    return xc * jax.lax.rsqrt(var + eps) * g + b


def _extract0_ct(img_ref):
    v = img_ref[0].reshape(3, 64, 4, 256)
    pieces = []
    for dy in range(4):
        t = v[:, :, dy, :].reshape(192, 256).T
        t = t.reshape(64, 4, 192)
        for dx in range(4):
            u = t[:, dx, :].T
            pieces.append(u.reshape(3, 64, 64))
    q = jnp.concatenate(pieces, axis=0)
    return q.reshape(48, 4096).astype(jnp.bfloat16)


def _stage0_body(pa_ref, pb_ref, pc_ref, wa_ref, wb_ref, wc_ref, eb_ref,
                 g1_ref, b1_ref, g2_ref, b2_ref, w1_ref, bb1_ref, w2_ref,
                 bb2_ref, o_ref, p_ref):
    tok = jnp.dot(wa_ref[...], _extract0_ct(pa_ref),
                  preferred_element_type=jnp.float32)
    tok = tok + jnp.dot(wb_ref[...], _extract0_ct(pb_ref),
                        preferred_element_type=jnp.float32)
    tok = tok + jnp.dot(wc_ref[...], _extract0_ct(pc_ref),
                        preferred_element_type=jnp.float32)
    tok = tok + eb_ref[...]
    tok = _ln_ct(tok, g1_ref[...], b1_ref[...])
    h = _ln_ct(tok, g2_ref[...], b2_ref[...])
    h = jnp.dot(w1_ref[...], h.astype(jnp.bfloat16),
                preferred_element_type=jnp.float32) + bb1_ref[...]
    h = jax.nn.gelu(h, approximate=True)
    out = (jnp.dot(w2_ref[...], h.astype(jnp.bfloat16),
                   preferred_element_type=jnp.float32) + bb2_ref[...] + tok)
    o_ref[...] = out.reshape(1, 32, 64, 64)
    p_ref[...] = _regroup(out.T, 64, 64, 32)


def _stage_body(p_in_ref, w_ref, eb_ref, g1_ref, b1_ref, g2_ref, b2_ref,
                w1_ref, bb1_ref, w2_ref, bb2_ref, o_ref, p_ref, *, hp, wp, c):
    out = _encoder_math(
        [p_in_ref[...]], [w_ref[...]],
        eb_ref[...], g1_ref[...], b1_ref[...], g2_ref[...], b2_ref[...],
        w1_ref[...], bb1_ref[...], w2_ref[...], bb2_ref[...])
    o_ref[...] = out.T.reshape(1, c, hp, wp)
    p_ref[...] = _regroup(out, hp, wp, c)


def _stage3_body(p_in_ref, w_ref, eb_ref, g1_ref, b1_ref, g2_ref, b2_ref,
                 w1_ref, bb1_ref, w2_ref, bb2_ref, o_ref, t_ref):
    out = _encoder_math(
        [p_in_ref[...]], [w_ref[...]],
        eb_ref[...], g1_ref[...], b1_ref[...], g2_ref[...], b2_ref[...],
        w1_ref[...], bb1_ref[...], w2_ref[...], bb2_ref[...])
    t = out.reshape(8, 64, 160)
    o_ref[...] = jnp.transpose(t, (0, 2, 1)).reshape(8, 160, 8, 8)
    t_ref[...] = out


def _full(shape):
    return pl.BlockSpec(shape, lambda *_, _s=shape: tuple(0 for _ in _s))


def _stage_params(st, c):
    h = 4 * c
    return (st["embed_b"].reshape(1, c),
            st["ln1_g"].reshape(1, c), st["ln1_b"].reshape(1, c),
            st["ln2_g"].reshape(1, c), st["ln2_b"].reshape(1, c),
            st["mlp_w1"], st["mlp_b1"].reshape(1, h),
            st["mlp_w2"], st["mlp_b2"].reshape(1, c))


def _stage_param_specs(kin, c):
    h = 4 * c
    return [_full((1, c)), _full((1, c)), _full((1, c)), _full((1, c)),
            _full((1, c)), _full((c, h)), _full((1, h)), _full((h, c)),
            _full((1, c))]


def _compiler_params():
    return pltpu.CompilerParams(
        dimension_semantics=("parallel", "arbitrary"),
        vmem_limit_bytes=_VMEM_LIMIT)


def _bilinear_matrix_np(out_size, in_size):
    dst = np.arange(out_size, dtype=np.float32)
    scale = in_size / out_size
    src = np.clip((dst + 0.5) * scale - 0.5, 0.0, in_size - 1)
    i0 = np.floor(src).astype(np.int32)
    i1 = np.minimum(i0 + 1, in_size - 1)
    lam = (src - i0).astype(np.float32)
    a = np.zeros((out_size, in_size), np.float32)
    rows = np.arange(out_size)
    a[rows, i0] += 1.0 - lam
    a[rows, i1] += lam
    return a


@functools.lru_cache(maxsize=None)
def _bilinear_factors(h, w, oh, ow):
    a = jnp.asarray(_bilinear_matrix_np(oh, h))
    bt = jnp.asarray(_bilinear_matrix_np(ow, w).T)
    return a, bt


def _head_body(f_ref, hw_ref, hb_ref, a_ref, bt_ref, o_ref):
    feat = f_ref[0]
    hv = jnp.sum(feat * hw_ref[...], axis=-1, keepdims=True) + hb_ref[0, 0]
    img = hv.reshape(8, 8)
    tmp = jnp.dot(img, bt_ref[...], preferred_element_type=jnp.float32)
    o_ref[0] = jnp.dot(a_ref[...], tmp, preferred_element_type=jnp.float32)


def _s0_perm(g):
    idx = np.empty((48,), np.int32)
    k = 0
    for dy in range(4):
        for dx in range(4):
            for ch in range(3):
                idx[k] = (dy * 4 + dx) * 9 + 3 * g + ch
                k += 1
    return idx


def kernel(x, y, xstack0, xstack1,
           s0_embed_w, s0_embed_b, s0_ln1_g, s0_ln1_b, s0_ln2_g, s0_ln2_b,
           s0_mlp_w1, s0_mlp_b1, s0_mlp_w2, s0_mlp_b2,
           s1_embed_w, s1_embed_b, s1_ln1_g, s1_ln1_b, s1_ln2_g, s1_ln2_b,
           s1_mlp_w1, s1_mlp_b1, s1_mlp_w2, s1_mlp_b2,
           s2_embed_w, s2_embed_b, s2_ln1_g, s2_ln1_b, s2_ln2_g, s2_ln2_b,
           s2_mlp_w1, s2_mlp_b1, s2_mlp_w2, s2_mlp_b2,
           s3_embed_w, s3_embed_b, s3_ln1_g, s3_ln1_b, s3_ln2_g, s3_ln2_b,
           s3_mlp_w1, s3_mlp_b1, s3_mlp_w2, s3_mlp_b2,
           head_w, head_b):
    stages = [
        dict(embed_w=s0_embed_w, embed_b=s0_embed_b, ln1_g=s0_ln1_g,
             ln1_b=s0_ln1_b, ln2_g=s0_ln2_g, ln2_b=s0_ln2_b,
             mlp_w1=s0_mlp_w1, mlp_b1=s0_mlp_b1, mlp_w2=s0_mlp_w2,
             mlp_b2=s0_mlp_b2),
        dict(embed_w=s1_embed_w, embed_b=s1_embed_b, ln1_g=s1_ln1_g,
             ln1_b=s1_ln1_b, ln2_g=s1_ln2_g, ln2_b=s1_ln2_b,
             mlp_w1=s1_mlp_w1, mlp_b1=s1_mlp_b1, mlp_w2=s1_mlp_w2,
             mlp_b2=s1_mlp_b2),
        dict(embed_w=s2_embed_w, embed_b=s2_embed_b, ln1_g=s2_ln1_g,
             ln1_b=s2_ln1_b, ln2_g=s2_ln2_g, ln2_b=s2_ln2_b,
             mlp_w1=s2_mlp_w1, mlp_b1=s2_mlp_b1, mlp_w2=s2_mlp_w2,
             mlp_b2=s2_mlp_b2),
        dict(embed_w=s3_embed_w, embed_b=s3_embed_b, ln1_g=s3_ln1_g,
             ln1_b=s3_ln1_b, ln2_g=s3_ln2_g, ln2_b=s3_ln2_b,
             mlp_w1=s3_mlp_w1, mlp_b1=s3_mlp_b1, mlp_w2=s3_mlp_w2,
             mlp_b2=s3_mlp_b2),
    ]

    wa = s0_embed_w[jnp.asarray(_s0_perm(0))].T
    wb = s0_embed_w[jnp.asarray(_s0_perm(1))].T
    wc = s0_embed_w[jnp.asarray(_s0_perm(2))].T

    st = stages[0]
    s0_params = (st["embed_b"].reshape(32, 1),
                 st["ln1_g"].reshape(32, 1), st["ln1_b"].reshape(32, 1),
                 st["ln2_g"].reshape(32, 1), st["ln2_b"].reshape(32, 1),
                 st["mlp_w1"].T, st["mlp_b1"].reshape(128, 1),
                 st["mlp_w2"].T, st["mlp_b2"].reshape(32, 1))
    s0_pspecs = [_full((32, 1)), _full((32, 1)), _full((32, 1)),
                 _full((32, 1)), _full((32, 1)), _full((128, 32)),
                 _full((128, 1)), _full((32, 128)), _full((32, 1))]
    out1, p1 = pl.pallas_call(
        _stage0_body,
        out_shape=(jax.ShapeDtypeStruct((B, 32, 64, 64), jnp.float32),
                   jax.ShapeDtypeStruct((B * 1024, 128), jnp.bfloat16)),
        grid=(2, 8),
        in_specs=[
            pl.BlockSpec((1, 3, 256, 256), lambda c, j: (c * 8 + j, 0, 0, 0)),
            pl.BlockSpec((1, 3, 256, 256), lambda c, j: (c * 8 + j, 0, 0, 0)),
            pl.BlockSpec((1, 3, 256, 256), lambda c, j: (c * 8 + j, 0, 0, 0)),
            _full((32, 48)), _full((32, 48)), _full((32, 48)),
        ] + s0_pspecs,
        out_specs=(pl.BlockSpec((1, 32, 64, 64), lambda c, j: (c * 8 + j, 0, 0, 0)),
                   pl.BlockSpec((1024, 128), lambda c, j: (c * 8 + j, 0))),
        compiler_params=_compiler_params(),
    )(xstack0, xstack1, y, wa, wb, wc, *s0_params)

    st = stages[1]
    out2, p2 = pl.pallas_call(
        functools.partial(_stage_body, hp=32, wp=32, c=64),
        out_shape=(jax.ShapeDtypeStruct((B, 64, 32, 32), jnp.float32),
                   jax.ShapeDtypeStruct((B * 256, 256), jnp.bfloat16)),
        grid=(2, 8),
        in_specs=[pl.BlockSpec((1024, 128), lambda c, j: (c * 8 + j, 0)),
                  _full((128, 64))] + _stage_param_specs(128, 64),
        out_specs=(pl.BlockSpec((1, 64, 32, 32), lambda c, j: (c * 8 + j, 0, 0, 0)),
                   pl.BlockSpec((256, 256), lambda c, j: (c * 8 + j, 0))),
        compiler_params=_compiler_params(),
    )(p1, st["embed_w"], *_stage_params(st, 64))

    st = stages[2]
    out3, p3 = pl.pallas_call(
        functools.partial(_stage_body, hp=16, wp=16, c=128),
        out_shape=(jax.ShapeDtypeStruct((B, 128, 16, 16), jnp.float32),
                   jax.ShapeDtypeStruct((B * 64, 512), jnp.bfloat16)),
        grid=(2, 8),
        in_specs=[pl.BlockSpec((256, 256), lambda c, j: (c * 8 + j, 0)),
                  _full((256, 128))] + _stage_param_specs(256, 128),
        out_specs=(pl.BlockSpec((1, 128, 16, 16), lambda c, j: (c * 8 + j, 0, 0, 0)),
                   pl.BlockSpec((64, 512), lambda c, j: (c * 8 + j, 0))),
        compiler_params=_compiler_params(),
    )(p2, st["embed_w"], *_stage_params(st, 128))

    st = stages[3]
    out4, tok4 = pl.pallas_call(
        _stage3_body,
        out_shape=(jax.ShapeDtypeStruct((B, 160, 8, 8), jnp.float32),
                   jax.ShapeDtypeStruct((B * 64, 160), jnp.float32)),
        grid=(2, 1),
        in_specs=[pl.BlockSpec((512, 512), lambda c, j: (c, 0)),
                  _full((512, 160))] + _stage_param_specs(512, 160),
        out_specs=(pl.BlockSpec((8, 160, 8, 8), lambda c, j: (c, 0, 0, 0)),
                   pl.BlockSpec((512, 160), lambda c, j: (c, 0))),
        compiler_params=_compiler_params(),
    )(p3, st["embed_w"], *_stage_params(st, 160))

    oh, ow = OUT_HW
    a_mat, bt_mat = _bilinear_factors(8, 8, oh, ow)
    feat = tok4.reshape(B, 64, 160)
    rgb = pl.pallas_call(
        _head_body,
        out_shape=jax.ShapeDtypeStruct((B, oh, ow), jnp.float32),
        grid=(2, 8),
        in_specs=[
            pl.BlockSpec((1, 64, 160), lambda c, j: (c * 8 + j, 0, 0)),
            _full((1, 160)), _full((1, 1)),
            _full((oh, 8)), _full((8, ow)),
        ],
        out_specs=pl.BlockSpec((1, oh, ow), lambda c, j: (c * 8 + j, 0, 0)),
        compiler_params=_compiler_params(),
    )(feat, head_w.reshape(1, 160), head_b.reshape(1, 1), a_mat, bt_mat)

    return rgb.reshape(B, 1, oh, ow), out1, out2, out3, out4


# speedup vs baseline: 28.7270x; 1.0659x over previous
import functools

import numpy as np
import jax
import jax.numpy as jnp
from jax.experimental import pallas as pl
from jax.experimental.pallas import tpu as pltpu

B = 16
EMBED_DIMS = (32, 64, 128, 160)
OUT_HW = (256, 256)
_VMEM_LIMIT = 48 * 1024 * 1024


def _ln_f32(x, g, b, eps=1e-5):
    mu = jnp.mean(x, axis=-1, keepdims=True)
    xc = x - mu
    var = jnp.mean(xc * xc, axis=-1, keepdims=True)
    return xc * jax.lax.rsqrt(var + eps) * g + b


def _encoder_math(xs, ws, eb, g1, b1, g2, b2, w1, bb1, w2, bb2):
    tok = jnp.dot(xs[0], ws[0], preferred_element_type=jnp.float32)
    for xv, wv in zip(xs[1:], ws[1:]):
        tok = tok + jnp.dot(xv, wv, preferred_element_type=jnp.float32)
    tok = tok + eb
    tok = _ln_f32(tok, g1, b1)
    h = _ln_f32(tok, g2, b2)
    h = jnp.dot(h.astype(jnp.bfloat16), w1, preferred_element_type=jnp.float32) + bb1
    h = jax.nn.gelu(h, approximate=True)
    return (jnp.dot(h.astype(jnp.bfloat16), w2,
                    preferred_element_type=jnp.float32) + bb2 + tok)


def _regroup(out, hp, wp, c):
    o = out.reshape(hp // 2, 2, wp // 2, 2, c)
    parts = [o[:, dy, :, dx, :] for dy in (0, 1) for dx in (0, 1)]
    p = jnp.concatenate(parts, axis=-1)
    return p.reshape((hp * wp) // 4, 4 * c).astype(jnp.bfloat16)


def _ln_ct(x, g, b, eps=1e-5):
    mu = jnp.mean(x, axis=0, keepdims=True)
    xc = x - mu
    var = jnp.mean(xc * xc, axis=0, keepdims=True)
    return xc * jax.lax.rsqrt(var + eps) * g + b


def _extract0_ct(img_ref):
    v = img_ref[0].reshape(3, 64, 4, 256)
    pieces = []
    for dy in range(4):
        t = v[:, :, dy, :].reshape(192, 256).T
        t = t.reshape(64, 4, 192)
        for dx in range(4):
            u = t[:, dx, :].T
            pieces.append(u.reshape(3, 64, 64))
    q = jnp.concatenate(pieces, axis=0)
    return q.reshape(48, 4096).astype(jnp.bfloat16)


def _stage0_body(pa_ref, pb_ref, pc_ref, wa_ref, wb_ref, wc_ref, eb_ref,
                 g1_ref, b1_ref, g2_ref, b2_ref, w1_ref, bb1_ref, w2_ref,
                 bb2_ref, o_ref, p_ref):
    tok = jnp.dot(wa_ref[...], _extract0_ct(pa_ref),
                  preferred_element_type=jnp.float32)
    tok = tok + jnp.dot(wb_ref[...], _extract0_ct(pb_ref),
                        preferred_element_type=jnp.float32)
    tok = tok + jnp.dot(wc_ref[...], _extract0_ct(pc_ref),
                        preferred_element_type=jnp.float32)
    tok = tok + eb_ref[...]
    tok = _ln_ct(tok, g1_ref[...], b1_ref[...])
    h = _ln_ct(tok, g2_ref[...], b2_ref[...])
    h = jnp.dot(w1_ref[...], h.astype(jnp.bfloat16),
                preferred_element_type=jnp.float32) + bb1_ref[...]
    h = jax.nn.gelu(h, approximate=True)
    out = (jnp.dot(w2_ref[...], h.astype(jnp.bfloat16),
                   preferred_element_type=jnp.float32) + bb2_ref[...] + tok)
    o_ref[...] = out.reshape(1, 32, 64, 64)
    p_ref[...] = _regroup(out.T, 64, 64, 32)


def _stage_body(p_in_ref, w_ref, eb_ref, g1_ref, b1_ref, g2_ref, b2_ref,
                w1_ref, bb1_ref, w2_ref, bb2_ref, o_ref, p_ref, *, hp, wp, c):
    out = _encoder_math(
        [p_in_ref[...]], [w_ref[...]],
        eb_ref[...], g1_ref[...], b1_ref[...], g2_ref[...], b2_ref[...],
        w1_ref[...], bb1_ref[...], w2_ref[...], bb2_ref[...])
    o_ref[...] = out.T.reshape(1, c, hp, wp)
    p_ref[...] = _regroup(out, hp, wp, c)


def _tail_body(p2_ref, w2_ref, s2_eb, s2_g1, s2_b1, s2_g2, s2_b2, s2_w1,
               s2_bb1, s2_w2, s2_bb2, w3_ref, s3_eb, s3_g1, s3_b1, s3_g2,
               s3_b2, s3_w1, s3_bb1, s3_w2, s3_bb2, hw_ref, hb_ref, a_ref,
               bt_ref, o3_ref, o4_ref, rgb_ref):
    out2 = _encoder_math(
        [p2_ref[...]], [w2_ref[...]], s2_eb[...], s2_g1[...], s2_b1[...],
        s2_g2[...], s2_b2[...], s2_w1[...], s2_bb1[...], s2_w2[...],
        s2_bb2[...])
    t3 = out2.reshape(8, 256, 128)
    o3_ref[...] = jnp.transpose(t3, (0, 2, 1)).reshape(8, 128, 16, 16)
    o = out2.reshape(8, 8, 2, 8, 2, 128)
    parts = [o[:, :, dy, :, dx, :] for dy in (0, 1) for dx in (0, 1)]
    p3 = jnp.concatenate(parts, axis=-1).reshape(512, 512).astype(jnp.bfloat16)
    out3 = _encoder_math(
        [p3], [w3_ref[...]], s3_eb[...], s3_g1[...], s3_b1[...], s3_g2[...],
        s3_b2[...], s3_w1[...], s3_bb1[...], s3_w2[...], s3_bb2[...])
    t4 = out3.reshape(8, 64, 160)
    o4_ref[...] = jnp.transpose(t4, (0, 2, 1)).reshape(8, 160, 8, 8)
    hv = jnp.sum(out3 * hw_ref[...], axis=-1, keepdims=True) + hb_ref[0, 0]
    tmp = jnp.dot(hv.reshape(64, 8), bt_ref[...],
                  preferred_element_type=jnp.float32)
    for b in range(8):
        rgb_ref[b] = jnp.dot(a_ref[...], tmp[8 * b:8 * b + 8, :],
                             preferred_element_type=jnp.float32)


def _full(shape):
    return pl.BlockSpec(shape, lambda *_, _s=shape: tuple(0 for _ in _s))


def _stage_params(st, c):
    h = 4 * c
    return (st["embed_b"].reshape(1, c),
            st["ln1_g"].reshape(1, c), st["ln1_b"].reshape(1, c),
            st["ln2_g"].reshape(1, c), st["ln2_b"].reshape(1, c),
            st["mlp_w1"], st["mlp_b1"].reshape(1, h),
            st["mlp_w2"], st["mlp_b2"].reshape(1, c))


def _stage_param_specs(kin, c):
    h = 4 * c
    return [_full((1, c)), _full((1, c)), _full((1, c)), _full((1, c)),
            _full((1, c)), _full((c, h)), _full((1, h)), _full((h, c)),
            _full((1, c))]


def _compiler_params():
    return pltpu.CompilerParams(
        dimension_semantics=("parallel", "arbitrary"),
        vmem_limit_bytes=_VMEM_LIMIT)


def _bilinear_matrix_np(out_size, in_size):
    dst = np.arange(out_size, dtype=np.float32)
    scale = in_size / out_size
    src = np.clip((dst + 0.5) * scale - 0.5, 0.0, in_size - 1)
    i0 = np.floor(src).astype(np.int32)
    i1 = np.minimum(i0 + 1, in_size - 1)
    lam = (src - i0).astype(np.float32)
    a = np.zeros((out_size, in_size), np.float32)
    rows = np.arange(out_size)
    a[rows, i0] += 1.0 - lam
    a[rows, i1] += lam
    return a


@functools.lru_cache(maxsize=None)
def _bilinear_factors(h, w, oh, ow):
    a = jnp.asarray(_bilinear_matrix_np(oh, h))
    bt = jnp.asarray(_bilinear_matrix_np(ow, w).T)
    return a, bt


def _head_body(f_ref, hw_ref, hb_ref, a_ref, bt_ref, o_ref):
    feat = f_ref[0]
    hv = jnp.sum(feat * hw_ref[...], axis=-1, keepdims=True) + hb_ref[0, 0]
    img = hv.reshape(8, 8)
    tmp = jnp.dot(img, bt_ref[...], preferred_element_type=jnp.float32)
    o_ref[0] = jnp.dot(a_ref[...], tmp, preferred_element_type=jnp.float32)


def _s0_perm(g):
    idx = np.empty((48,), np.int32)
    k = 0
    for dy in range(4):
        for dx in range(4):
            for ch in range(3):
                idx[k] = (dy * 4 + dx) * 9 + 3 * g + ch
                k += 1
    return idx


def kernel(x, y, xstack0, xstack1,
           s0_embed_w, s0_embed_b, s0_ln1_g, s0_ln1_b, s0_ln2_g, s0_ln2_b,
           s0_mlp_w1, s0_mlp_b1, s0_mlp_w2, s0_mlp_b2,
           s1_embed_w, s1_embed_b, s1_ln1_g, s1_ln1_b, s1_ln2_g, s1_ln2_b,
           s1_mlp_w1, s1_mlp_b1, s1_mlp_w2, s1_mlp_b2,
           s2_embed_w, s2_embed_b, s2_ln1_g, s2_ln1_b, s2_ln2_g, s2_ln2_b,
           s2_mlp_w1, s2_mlp_b1, s2_mlp_w2, s2_mlp_b2,
           s3_embed_w, s3_embed_b, s3_ln1_g, s3_ln1_b, s3_ln2_g, s3_ln2_b,
           s3_mlp_w1, s3_mlp_b1, s3_mlp_w2, s3_mlp_b2,
           head_w, head_b):
    stages = [
        dict(embed_w=s0_embed_w, embed_b=s0_embed_b, ln1_g=s0_ln1_g,
             ln1_b=s0_ln1_b, ln2_g=s0_ln2_g, ln2_b=s0_ln2_b,
             mlp_w1=s0_mlp_w1, mlp_b1=s0_mlp_b1, mlp_w2=s0_mlp_w2,
             mlp_b2=s0_mlp_b2),
        dict(embed_w=s1_embed_w, embed_b=s1_embed_b, ln1_g=s1_ln1_g,
             ln1_b=s1_ln1_b, ln2_g=s1_ln2_g, ln2_b=s1_ln2_b,
             mlp_w1=s1_mlp_w1, mlp_b1=s1_mlp_b1, mlp_w2=s1_mlp_w2,
             mlp_b2=s1_mlp_b2),
        dict(embed_w=s2_embed_w, embed_b=s2_embed_b, ln1_g=s2_ln1_g,
             ln1_b=s2_ln1_b, ln2_g=s2_ln2_g, ln2_b=s2_ln2_b,
             mlp_w1=s2_mlp_w1, mlp_b1=s2_mlp_b1, mlp_w2=s2_mlp_w2,
             mlp_b2=s2_mlp_b2),
        dict(embed_w=s3_embed_w, embed_b=s3_embed_b, ln1_g=s3_ln1_g,
             ln1_b=s3_ln1_b, ln2_g=s3_ln2_g, ln2_b=s3_ln2_b,
             mlp_w1=s3_mlp_w1, mlp_b1=s3_mlp_b1, mlp_w2=s3_mlp_w2,
             mlp_b2=s3_mlp_b2),
    ]

    wa = s0_embed_w[jnp.asarray(_s0_perm(0))].T
    wb = s0_embed_w[jnp.asarray(_s0_perm(1))].T
    wc = s0_embed_w[jnp.asarray(_s0_perm(2))].T

    st = stages[0]
    s0_params = (st["embed_b"].reshape(32, 1),
                 st["ln1_g"].reshape(32, 1), st["ln1_b"].reshape(32, 1),
                 st["ln2_g"].reshape(32, 1), st["ln2_b"].reshape(32, 1),
                 st["mlp_w1"].T, st["mlp_b1"].reshape(128, 1),
                 st["mlp_w2"].T, st["mlp_b2"].reshape(32, 1))
    s0_pspecs = [_full((32, 1)), _full((32, 1)), _full((32, 1)),
                 _full((32, 1)), _full((32, 1)), _full((128, 32)),
                 _full((128, 1)), _full((32, 128)), _full((32, 1))]
    out1, p1 = pl.pallas_call(
        _stage0_body,
        out_shape=(jax.ShapeDtypeStruct((B, 32, 64, 64), jnp.float32),
                   jax.ShapeDtypeStruct((B * 1024, 128), jnp.bfloat16)),
        grid=(2, 8),
        in_specs=[
            pl.BlockSpec((1, 3, 256, 256), lambda c, j: (c * 8 + j, 0, 0, 0)),
            pl.BlockSpec((1, 3, 256, 256), lambda c, j: (c * 8 + j, 0, 0, 0)),
            pl.BlockSpec((1, 3, 256, 256), lambda c, j: (c * 8 + j, 0, 0, 0)),
            _full((32, 48)), _full((32, 48)), _full((32, 48)),
        ] + s0_pspecs,
        out_specs=(pl.BlockSpec((1, 32, 64, 64), lambda c, j: (c * 8 + j, 0, 0, 0)),
                   pl.BlockSpec((1024, 128), lambda c, j: (c * 8 + j, 0))),
        compiler_params=_compiler_params(),
    )(xstack0, xstack1, y, wa, wb, wc, *s0_params)

    st = stages[1]
    out2, p2 = pl.pallas_call(
        functools.partial(_stage_body, hp=32, wp=32, c=64),
        out_shape=(jax.ShapeDtypeStruct((B, 64, 32, 32), jnp.float32),
                   jax.ShapeDtypeStruct((B * 256, 256), jnp.bfloat16)),
        grid=(2, 8),
        in_specs=[pl.BlockSpec((1024, 128), lambda c, j: (c * 8 + j, 0)),
                  _full((128, 64))] + _stage_param_specs(128, 64),
        out_specs=(pl.BlockSpec((1, 64, 32, 32), lambda c, j: (c * 8 + j, 0, 0, 0)),
                   pl.BlockSpec((256, 256), lambda c, j: (c * 8 + j, 0))),
        compiler_params=_compiler_params(),
    )(p1, st["embed_w"], *_stage_params(st, 64))

    oh, ow = OUT_HW
    a_mat, bt_mat = _bilinear_factors(8, 8, oh, ow)
    st2, st3 = stages[2], stages[3]
    out3, out4, rgb = pl.pallas_call(
        _tail_body,
        out_shape=(jax.ShapeDtypeStruct((B, 128, 16, 16), jnp.float32),
                   jax.ShapeDtypeStruct((B, 160, 8, 8), jnp.float32),
                   jax.ShapeDtypeStruct((B, oh, ow), jnp.float32)),
        grid=(2, 1),
        in_specs=[pl.BlockSpec((2048, 256), lambda c, j: (c, 0)),
                  _full((256, 128))] + _stage_param_specs(256, 128)
                 + [_full((512, 160))] + _stage_param_specs(512, 160)
                 + [_full((1, 160)), _full((1, 1)),
                    _full((oh, 8)), _full((8, ow))],
        out_specs=(pl.BlockSpec((8, 128, 16, 16), lambda c, j: (c, 0, 0, 0)),
                   pl.BlockSpec((8, 160, 8, 8), lambda c, j: (c, 0, 0, 0)),
                   pl.BlockSpec((8, oh, ow), lambda c, j: (c, 0, 0))),
        compiler_params=_compiler_params(),
    )(p2, st2["embed_w"], *_stage_params(st2, 128), st3["embed_w"],
      *_stage_params(st3, 160), head_w.reshape(1, 160),
      head_b.reshape(1, 1), a_mat, bt_mat)

    return rgb.reshape(B, 1, oh, ow), out1, out2, out3, out4


# speedup vs baseline: 29.0982x; 1.0129x over previous
import functools

import numpy as np
import jax
import jax.numpy as jnp
from jax.experimental import pallas as pl
from jax.experimental.pallas import tpu as pltpu

B = 16
EMBED_DIMS = (32, 64, 128, 160)
OUT_HW = (256, 256)
_VMEM_LIMIT = 48 * 1024 * 1024


def _ln_f32(x, g, b, eps=1e-5):
    mu = jnp.mean(x, axis=-1, keepdims=True)
    xc = x - mu
    var = jnp.mean(xc * xc, axis=-1, keepdims=True)
    return xc * jax.lax.rsqrt(var + eps) * g + b


def _encoder_math(xs, ws, eb, g1, b1, g2, b2, w1, bb1, w2, bb2):
    tok = jnp.dot(xs[0], ws[0], preferred_element_type=jnp.float32)
    for xv, wv in zip(xs[1:], ws[1:]):
        tok = tok + jnp.dot(xv, wv, preferred_element_type=jnp.float32)
    tok = tok + eb
    tok = _ln_f32(tok, g1, b1)
    h = _ln_f32(tok, g2, b2)
    h = jnp.dot(h.astype(jnp.bfloat16), w1, preferred_element_type=jnp.float32) + bb1
    h = jax.nn.gelu(h, approximate=True)
    return (jnp.dot(h.astype(jnp.bfloat16), w2,
                    preferred_element_type=jnp.float32) + bb2 + tok)


def _regroup(out, hp, wp, c):
    o = out.reshape(hp // 2, 2, wp // 2, 2, c)
    parts = [o[:, dy, :, dx, :] for dy in (0, 1) for dx in (0, 1)]
    p = jnp.concatenate(parts, axis=-1)
    return p.reshape((hp * wp) // 4, 4 * c).astype(jnp.bfloat16)


def _ln_ct(x, g, b, eps=1e-5):
    mu = jnp.mean(x, axis=0, keepdims=True)
    xc = x - mu
    var = jnp.mean(xc * xc, axis=0, keepdims=True)
    return xc * jax.lax.rsqrt(var + eps) * g + b


def _extract0_ct(img_ref):
    v = img_ref[0].reshape(3, 64, 4, 256)
    pieces = []
    for dy in range(4):
        t = v[:, :, dy, :].reshape(192, 256).T
        t = t.reshape(64, 4, 192)
        for dx in range(4):
            u = t[:, dx, :].T
            pieces.append(u.reshape(3, 64, 64))
    q = jnp.concatenate(pieces, axis=0)
    return q.reshape(48, 4096).astype(jnp.bfloat16)


def _stage0_body(pa_ref, pb_ref, pc_ref, wa_ref, wb_ref, wc_ref, eb_ref,
                 g1_ref, b1_ref, g2_ref, b2_ref, w1_ref, bb1_ref, w2_ref,
                 bb2_ref, o_ref):
    tok = jnp.dot(wa_ref[...], _extract0_ct(pa_ref),
                  preferred_element_type=jnp.float32)
    tok = tok + jnp.dot(wb_ref[...], _extract0_ct(pb_ref),
                        preferred_element_type=jnp.float32)
    tok = tok + jnp.dot(wc_ref[...], _extract0_ct(pc_ref),
                        preferred_element_type=jnp.float32)
    tok = tok + eb_ref[...]
    tok = _ln_ct(tok, g1_ref[...], b1_ref[...])
    h = _ln_ct(tok, g2_ref[...], b2_ref[...])
    h = jnp.dot(w1_ref[...], h.astype(jnp.bfloat16),
                preferred_element_type=jnp.float32) + bb1_ref[...]
    h = jax.nn.gelu(h, approximate=True)
    out = (jnp.dot(w2_ref[...], h.astype(jnp.bfloat16),
                   preferred_element_type=jnp.float32) + bb2_ref[...] + tok)
    o_ref[...] = out.reshape(1, 32, 64, 64)


def _extract_ct(x, cin, hp, wp):
    v = x.reshape(cin, hp, 2, 2 * wp)
    pieces = []
    for dy in range(2):
        r = v[:, :, dy, :].reshape(cin * hp, 2 * wp).T
        r = r.reshape(wp, 2, cin * hp)
        for dx in range(2):
            u = r[:, dx, :].T
            pieces.append(u.reshape(cin, hp, wp))
    q = jnp.concatenate(pieces, axis=0)
    return q.reshape(4 * cin, hp * wp).astype(jnp.bfloat16)


def _stage1_body(x_ref, w_ref, eb_ref, g1_ref,
                 b1_ref, g2_ref, b2_ref, w1_ref, bb1_ref, w2_ref, bb2_ref,
                 o_ref, p_ref):
    q = _extract_ct(x_ref[0], 32, 32, 32)
    tok = jnp.dot(w_ref[...], q, preferred_element_type=jnp.float32)
    tok = tok + eb_ref[...]
    tok = _ln_ct(tok, g1_ref[...], b1_ref[...])
    h = _ln_ct(tok, g2_ref[...], b2_ref[...])
    h = jnp.dot(w1_ref[...], h.astype(jnp.bfloat16),
                preferred_element_type=jnp.float32) + bb1_ref[...]
    h = jax.nn.gelu(h, approximate=True)
    out = (jnp.dot(w2_ref[...], h.astype(jnp.bfloat16),
                   preferred_element_type=jnp.float32) + bb2_ref[...] + tok)
    o_ref[...] = out.reshape(1, 64, 32, 32)
    p_ref[...] = _regroup(out.T, 32, 32, 64)


def _tail_body(p2_ref, w2_ref, s2_eb, s2_g1, s2_b1, s2_g2, s2_b2, s2_w1,
               s2_bb1, s2_w2, s2_bb2, w3_ref, s3_eb, s3_g1, s3_b1, s3_g2,
               s3_b2, s3_w1, s3_bb1, s3_w2, s3_bb2, hw_ref, hb_ref, a_ref,
               bt_ref, o3_ref, o4_ref, rgb_ref):
    out2 = _encoder_math(
        [p2_ref[...]], [w2_ref[...]], s2_eb[...], s2_g1[...], s2_b1[...],
        s2_g2[...], s2_b2[...], s2_w1[...], s2_bb1[...], s2_w2[...],
        s2_bb2[...])
    t3 = out2.reshape(8, 256, 128)
    o3_ref[...] = jnp.transpose(t3, (0, 2, 1)).reshape(8, 128, 16, 16)
    o = out2.reshape(8, 8, 2, 8, 2, 128)
    parts = [o[:, :, dy, :, dx, :] for dy in (0, 1) for dx in (0, 1)]
    p3 = jnp.concatenate(parts, axis=-1).reshape(512, 512).astype(jnp.bfloat16)
    out3 = _encoder_math(
        [p3], [w3_ref[...]], s3_eb[...], s3_g1[...], s3_b1[...], s3_g2[...],
        s3_b2[...], s3_w1[...], s3_bb1[...], s3_w2[...], s3_bb2[...])
    t4 = out3.reshape(8, 64, 160)
    o4_ref[...] = jnp.transpose(t4, (0, 2, 1)).reshape(8, 160, 8, 8)
    hv = jnp.sum(out3 * hw_ref[...], axis=-1, keepdims=True) + hb_ref[0, 0]
    tmp = jnp.dot(hv.reshape(64, 8), bt_ref[...],
                  preferred_element_type=jnp.float32)
    for b in range(8):
        rgb_ref[b] = jnp.dot(a_ref[...], tmp[8 * b:8 * b + 8, :],
                             preferred_element_type=jnp.float32)


def _full(shape):
    return pl.BlockSpec(shape, lambda *_, _s=shape: tuple(0 for _ in _s))


def _stage_params(st, c):
    h = 4 * c
    return (st["embed_b"].reshape(1, c),
            st["ln1_g"].reshape(1, c), st["ln1_b"].reshape(1, c),
            st["ln2_g"].reshape(1, c), st["ln2_b"].reshape(1, c),
            st["mlp_w1"], st["mlp_b1"].reshape(1, h),
            st["mlp_w2"], st["mlp_b2"].reshape(1, c))


def _stage_param_specs(kin, c):
    h = 4 * c
    return [_full((1, c)), _full((1, c)), _full((1, c)), _full((1, c)),
            _full((1, c)), _full((c, h)), _full((1, h)), _full((h, c)),
            _full((1, c))]


def _compiler_params():
    return pltpu.CompilerParams(
        dimension_semantics=("parallel", "arbitrary"),
        vmem_limit_bytes=_VMEM_LIMIT)


def _bilinear_matrix_np(out_size, in_size):
    dst = np.arange(out_size, dtype=np.float32)
    scale = in_size / out_size
    src = np.clip((dst + 0.5) * scale - 0.5, 0.0, in_size - 1)
    i0 = np.floor(src).astype(np.int32)
    i1 = np.minimum(i0 + 1, in_size - 1)
    lam = (src - i0).astype(np.float32)
    a = np.zeros((out_size, in_size), np.float32)
    rows = np.arange(out_size)
    a[rows, i0] += 1.0 - lam
    a[rows, i1] += lam
    return a


@functools.lru_cache(maxsize=None)
def _bilinear_factors(h, w, oh, ow):
    a = jnp.asarray(_bilinear_matrix_np(oh, h))
    bt = jnp.asarray(_bilinear_matrix_np(ow, w).T)
    return a, bt


def _head_body(f_ref, hw_ref, hb_ref, a_ref, bt_ref, o_ref):
    feat = f_ref[0]
    hv = jnp.sum(feat * hw_ref[...], axis=-1, keepdims=True) + hb_ref[0, 0]
    img = hv.reshape(8, 8)
    tmp = jnp.dot(img, bt_ref[...], preferred_element_type=jnp.float32)
    o_ref[0] = jnp.dot(a_ref[...], tmp, preferred_element_type=jnp.float32)


def _s0_perm(g):
    idx = np.empty((48,), np.int32)
    k = 0
    for dy in range(4):
        for dx in range(4):
            for ch in range(3):
                idx[k] = (dy * 4 + dx) * 9 + 3 * g + ch
                k += 1
    return idx


def kernel(x, y, xstack0, xstack1,
           s0_embed_w, s0_embed_b, s0_ln1_g, s0_ln1_b, s0_ln2_g, s0_ln2_b,
           s0_mlp_w1, s0_mlp_b1, s0_mlp_w2, s0_mlp_b2,
           s1_embed_w, s1_embed_b, s1_ln1_g, s1_ln1_b, s1_ln2_g, s1_ln2_b,
           s1_mlp_w1, s1_mlp_b1, s1_mlp_w2, s1_mlp_b2,
           s2_embed_w, s2_embed_b, s2_ln1_g, s2_ln1_b, s2_ln2_g, s2_ln2_b,
           s2_mlp_w1, s2_mlp_b1, s2_mlp_w2, s2_mlp_b2,
           s3_embed_w, s3_embed_b, s3_ln1_g, s3_ln1_b, s3_ln2_g, s3_ln2_b,
           s3_mlp_w1, s3_mlp_b1, s3_mlp_w2, s3_mlp_b2,
           head_w, head_b):
    stages = [
        dict(embed_w=s0_embed_w, embed_b=s0_embed_b, ln1_g=s0_ln1_g,
             ln1_b=s0_ln1_b, ln2_g=s0_ln2_g, ln2_b=s0_ln2_b,
             mlp_w1=s0_mlp_w1, mlp_b1=s0_mlp_b1, mlp_w2=s0_mlp_w2,
             mlp_b2=s0_mlp_b2),
        dict(embed_w=s1_embed_w, embed_b=s1_embed_b, ln1_g=s1_ln1_g,
             ln1_b=s1_ln1_b, ln2_g=s1_ln2_g, ln2_b=s1_ln2_b,
             mlp_w1=s1_mlp_w1, mlp_b1=s1_mlp_b1, mlp_w2=s1_mlp_w2,
             mlp_b2=s1_mlp_b2),
        dict(embed_w=s2_embed_w, embed_b=s2_embed_b, ln1_g=s2_ln1_g,
             ln1_b=s2_ln1_b, ln2_g=s2_ln2_g, ln2_b=s2_ln2_b,
             mlp_w1=s2_mlp_w1, mlp_b1=s2_mlp_b1, mlp_w2=s2_mlp_w2,
             mlp_b2=s2_mlp_b2),
        dict(embed_w=s3_embed_w, embed_b=s3_embed_b, ln1_g=s3_ln1_g,
             ln1_b=s3_ln1_b, ln2_g=s3_ln2_g, ln2_b=s3_ln2_b,
             mlp_w1=s3_mlp_w1, mlp_b1=s3_mlp_b1, mlp_w2=s3_mlp_w2,
             mlp_b2=s3_mlp_b2),
    ]

    wa = s0_embed_w[jnp.asarray(_s0_perm(0))].T
    wb = s0_embed_w[jnp.asarray(_s0_perm(1))].T
    wc = s0_embed_w[jnp.asarray(_s0_perm(2))].T

    st = stages[0]
    s0_params = (st["embed_b"].reshape(32, 1),
                 st["ln1_g"].reshape(32, 1), st["ln1_b"].reshape(32, 1),
                 st["ln2_g"].reshape(32, 1), st["ln2_b"].reshape(32, 1),
                 st["mlp_w1"].T, st["mlp_b1"].reshape(128, 1),
                 st["mlp_w2"].T, st["mlp_b2"].reshape(32, 1))
    s0_pspecs = [_full((32, 1)), _full((32, 1)), _full((32, 1)),
                 _full((32, 1)), _full((32, 1)), _full((128, 32)),
                 _full((128, 1)), _full((32, 128)), _full((32, 1))]
    out1 = pl.pallas_call(
        _stage0_body,
        out_shape=jax.ShapeDtypeStruct((B, 32, 64, 64), jnp.float32),
        grid=(2, 8),
        in_specs=[
            pl.BlockSpec((1, 3, 256, 256), lambda c, j: (c * 8 + j, 0, 0, 0)),
            pl.BlockSpec((1, 3, 256, 256), lambda c, j: (c * 8 + j, 0, 0, 0)),
            pl.BlockSpec((1, 3, 256, 256), lambda c, j: (c * 8 + j, 0, 0, 0)),
            _full((32, 48)), _full((32, 48)), _full((32, 48)),
        ] + s0_pspecs,
        out_specs=pl.BlockSpec((1, 32, 64, 64), lambda c, j: (c * 8 + j, 0, 0, 0)),
        compiler_params=_compiler_params(),
    )(xstack0, xstack1, y, wa, wb, wc, *s0_params)

    st = stages[1]
    w1T = st["embed_w"].T
    s1_params = (st["embed_b"].reshape(64, 1),
                 st["ln1_g"].reshape(64, 1), st["ln1_b"].reshape(64, 1),
                 st["ln2_g"].reshape(64, 1), st["ln2_b"].reshape(64, 1),
                 st["mlp_w1"].T, st["mlp_b1"].reshape(256, 1),
                 st["mlp_w2"].T, st["mlp_b2"].reshape(64, 1))
    s1_pspecs = [_full((64, 1)), _full((64, 1)), _full((64, 1)),
                 _full((64, 1)), _full((64, 1)), _full((256, 64)),
                 _full((256, 1)), _full((64, 256)), _full((64, 1))]
    out2, p2 = pl.pallas_call(
        _stage1_body,
        out_shape=(jax.ShapeDtypeStruct((B, 64, 32, 32), jnp.float32),
                   jax.ShapeDtypeStruct((B * 256, 256), jnp.bfloat16)),
        grid=(2, 8),
        in_specs=[pl.BlockSpec((1, 32, 64, 64), lambda c, j: (c * 8 + j, 0, 0, 0)),
                  _full((64, 128))] + s1_pspecs,
        out_specs=(pl.BlockSpec((1, 64, 32, 32), lambda c, j: (c * 8 + j, 0, 0, 0)),
                   pl.BlockSpec((256, 256), lambda c, j: (c * 8 + j, 0))),
        compiler_params=_compiler_params(),
    )(out1, w1T, *s1_params)

    oh, ow = OUT_HW
    a_mat, bt_mat = _bilinear_factors(8, 8, oh, ow)
    st2, st3 = stages[2], stages[3]
    out3, out4, rgb = pl.pallas_call(
        _tail_body,
        out_shape=(jax.ShapeDtypeStruct((B, 128, 16, 16), jnp.float32),
                   jax.ShapeDtypeStruct((B, 160, 8, 8), jnp.float32),
                   jax.ShapeDtypeStruct((B, oh, ow), jnp.float32)),
        grid=(2, 1),
        in_specs=[pl.BlockSpec((2048, 256), lambda c, j: (c, 0)),
                  _full((256, 128))] + _stage_param_specs(256, 128)
                 + [_full((512, 160))] + _stage_param_specs(512, 160)
                 + [_full((1, 160)), _full((1, 1)),
                    _full((oh, 8)), _full((8, ow))],
        out_specs=(pl.BlockSpec((8, 128, 16, 16), lambda c, j: (c, 0, 0, 0)),
                   pl.BlockSpec((8, 160, 8, 8), lambda c, j: (c, 0, 0, 0)),
                   pl.BlockSpec((8, oh, ow), lambda c, j: (c, 0, 0))),
        compiler_params=_compiler_params(),
    )(p2, st2["embed_w"], *_stage_params(st2, 128), st3["embed_w"],
      *_stage_params(st3, 160), head_w.reshape(1, 160),
      head_b.reshape(1, 1), a_mat, bt_mat)

    return rgb.reshape(B, 1, oh, ow), out1, out2, out3, out4


# speedup vs baseline: 29.1390x; 1.0014x over previous
import functools

import numpy as np
import jax
import jax.numpy as jnp
from jax.experimental import pallas as pl
from jax.experimental.pallas import tpu as pltpu

B = 16
EMBED_DIMS = (32, 64, 128, 160)
OUT_HW = (256, 256)
_VMEM_LIMIT = 48 * 1024 * 1024


def _ln_f32(x, g, b, eps=1e-5):
    mu = jnp.mean(x, axis=-1, keepdims=True)
    xc = x - mu
    var = jnp.mean(xc * xc, axis=-1, keepdims=True)
    return xc * jax.lax.rsqrt(var + eps) * g + b


def _encoder_math(xs, ws, eb, g1, b1, g2, b2, w1, bb1, w2, bb2):
    tok = jnp.dot(xs[0], ws[0], preferred_element_type=jnp.float32)
    for xv, wv in zip(xs[1:], ws[1:]):
        tok = tok + jnp.dot(xv, wv, preferred_element_type=jnp.float32)
    tok = tok + eb
    tok = _ln_f32(tok, g1, b1)
    h = _ln_f32(tok, g2, b2)
    h = jnp.dot(h.astype(jnp.bfloat16), w1, preferred_element_type=jnp.float32) + bb1
    h = jax.nn.gelu(h, approximate=True)
    return (jnp.dot(h.astype(jnp.bfloat16), w2,
                    preferred_element_type=jnp.float32) + bb2 + tok)


def _regroup(out, hp, wp, c):
    o = out.reshape(hp // 2, 2, wp // 2, 2, c)
    parts = [o[:, dy, :, dx, :] for dy in (0, 1) for dx in (0, 1)]
    p = jnp.concatenate(parts, axis=-1)
    return p.reshape((hp * wp) // 4, 4 * c).astype(jnp.bfloat16)


def _ln_ct(x, g, b, eps=1e-5):
    mu = jnp.mean(x, axis=0, keepdims=True)
    xc = x - mu
    var = jnp.mean(xc * xc, axis=0, keepdims=True)
    return xc * jax.lax.rsqrt(var + eps) * g + b


def _extract0_ct(img_ref):
    v = img_ref[0].reshape(3, 64, 4, 256)
    pieces = []
    for dy in range(4):
        t = v[:, :, dy, :].reshape(192, 256).T
        t = t.reshape(64, 4, 192)
        for dx in range(4):
            u = t[:, dx, :].T
            pieces.append(u.reshape(3, 64, 64))
    q = jnp.concatenate(pieces, axis=0)
    return q.reshape(48, 4096).astype(jnp.bfloat16)


def _stage0_body(pa_ref, pb_ref, pc_ref, wa_ref, wb_ref, wc_ref, eb_ref,
                 g1_ref, b1_ref, g2_ref, b2_ref, w1_ref, bb1_ref, w2_ref,
                 bb2_ref, o_ref):
    tok = jnp.dot(wa_ref[...], _extract0_ct(pa_ref),
                  preferred_element_type=jnp.float32)
    tok = tok + jnp.dot(wb_ref[...], _extract0_ct(pb_ref),
                        preferred_element_type=jnp.float32)
    tok = tok + jnp.dot(wc_ref[...], _extract0_ct(pc_ref),
                        preferred_element_type=jnp.float32)
    tok = tok + eb_ref[...]
    tok = _ln_ct(tok, g1_ref[...], b1_ref[...])
    h = _ln_ct(tok, g2_ref[...], b2_ref[...])
    h = jnp.dot(w1_ref[...], h.astype(jnp.bfloat16),
                preferred_element_type=jnp.float32) + bb1_ref[...]
    h = jax.nn.gelu(h, approximate=True)
    out = (jnp.dot(w2_ref[...], h.astype(jnp.bfloat16),
                   preferred_element_type=jnp.float32) + bb2_ref[...] + tok)
    o_ref[...] = out.reshape(1, 32, 64, 64)


def _extract_ct(x, cin, hp, wp):
    v = x.reshape(cin, hp, 2, 2 * wp)
    pieces = []
    for dy in range(2):
        r = v[:, :, dy, :].reshape(cin * hp, 2 * wp).T
        r = r.reshape(wp, 2, cin * hp)
        for dx in range(2):
            u = r[:, dx, :].T
            pieces.append(u.reshape(cin, hp, wp))
    q = jnp.concatenate(pieces, axis=0)
    return q.reshape(4 * cin, hp * wp).astype(jnp.bfloat16)


def _stage1_body(x_ref, w_ref, eb_ref, g1_ref,
                 b1_ref, g2_ref, b2_ref, w1_ref, bb1_ref, w2_ref, bb2_ref,
                 o_ref, p_ref):
    q = _extract_ct(x_ref[0], 32, 32, 32)
    tok = jnp.dot(w_ref[...], q, preferred_element_type=jnp.float32)
    tok = tok + eb_ref[...]
    tok = _ln_ct(tok, g1_ref[...], b1_ref[...])
    h = _ln_ct(tok, g2_ref[...], b2_ref[...])
    h = jnp.dot(w1_ref[...], h.astype(jnp.bfloat16),
                preferred_element_type=jnp.float32) + bb1_ref[...]
    h = jax.nn.gelu(h, approximate=True)
    out = (jnp.dot(w2_ref[...], h.astype(jnp.bfloat16),
                   preferred_element_type=jnp.float32) + bb2_ref[...] + tok)
    o_ref[...] = out.reshape(1, 64, 32, 32)
    p_ref[...] = _regroup(out.T, 32, 32, 64)


def _tail_body(p2_ref, w2_ref, s2_eb, s2_g1, s2_b1, s2_g2, s2_b2, s2_w1,
               s2_bb1, s2_w2, s2_bb2, w3_ref, s3_eb, s3_g1, s3_b1, s3_g2,
               s3_b2, s3_w1, s3_bb1, s3_w2, s3_bb2, hw_ref, hb_ref, a_ref,
               bt_ref, o3_ref, o4_ref, rgb_ref):
    out2 = _encoder_math(
        [p2_ref[...]], [w2_ref[...]], s2_eb[...], s2_g1[...], s2_b1[...],
        s2_g2[...], s2_b2[...], s2_w1[...], s2_bb1[...], s2_w2[...],
        s2_bb2[...])
    t3 = out2.reshape(8, 256, 128)
    o3_ref[...] = jnp.transpose(t3, (0, 2, 1)).reshape(8, 128, 16, 16)
    o = out2.reshape(8, 8, 2, 8, 2, 128)
    parts = [o[:, :, dy, :, dx, :] for dy in (0, 1) for dx in (0, 1)]
    p3 = jnp.concatenate(parts, axis=-1).reshape(512, 512).astype(jnp.bfloat16)
    out3 = _encoder_math(
        [p3], [w3_ref[...]], s3_eb[...], s3_g1[...], s3_b1[...], s3_g2[...],
        s3_b2[...], s3_w1[...], s3_bb1[...], s3_w2[...], s3_bb2[...])
    t4 = out3.reshape(8, 64, 160)
    o4_ref[...] = jnp.transpose(t4, (0, 2, 1)).reshape(8, 160, 8, 8)
    hv = jnp.sum(out3 * hw_ref[...], axis=-1, keepdims=True) + hb_ref[0, 0]
    tmp = jnp.dot(hv.reshape(64, 8), bt_ref[...],
                  preferred_element_type=jnp.float32)
    for b in range(8):
        rgb_ref[b] = jnp.dot(a_ref[...], tmp[8 * b:8 * b + 8, :],
                             preferred_element_type=jnp.float32)


def _full(shape):
    return pl.BlockSpec(shape, lambda *_, _s=shape: tuple(0 for _ in _s))


def _stage_params(st, c):
    h = 4 * c
    return (st["embed_b"].reshape(1, c),
            st["ln1_g"].reshape(1, c), st["ln1_b"].reshape(1, c),
            st["ln2_g"].reshape(1, c), st["ln2_b"].reshape(1, c),
            st["mlp_w1"], st["mlp_b1"].reshape(1, h),
            st["mlp_w2"], st["mlp_b2"].reshape(1, c))


def _stage_param_specs(kin, c):
    h = 4 * c
    return [_full((1, c)), _full((1, c)), _full((1, c)), _full((1, c)),
            _full((1, c)), _full((c, h)), _full((1, h)), _full((h, c)),
            _full((1, c))]


def _compiler_params():
    return pltpu.CompilerParams(
        dimension_semantics=("parallel", "arbitrary"),
        vmem_limit_bytes=_VMEM_LIMIT)


def _bilinear_matrix_np(out_size, in_size):
    dst = np.arange(out_size, dtype=np.float32)
    scale = in_size / out_size
    src = np.clip((dst + 0.5) * scale - 0.5, 0.0, in_size - 1)
    i0 = np.floor(src).astype(np.int32)
    i1 = np.minimum(i0 + 1, in_size - 1)
    lam = (src - i0).astype(np.float32)
    a = np.zeros((out_size, in_size), np.float32)
    rows = np.arange(out_size)
    a[rows, i0] += 1.0 - lam
    a[rows, i1] += lam
    return a


@functools.lru_cache(maxsize=None)
def _bilinear_factors(h, w, oh, ow):
    a = jnp.asarray(_bilinear_matrix_np(oh, h))
    bt = jnp.asarray(_bilinear_matrix_np(ow, w).T)
    return a, bt


def _s0_perm(g):
    idx = np.empty((48,), np.int32)
    k = 0
    for dy in range(4):
        for dx in range(4):
            for ch in range(3):
                idx[k] = (dy * 4 + dx) * 9 + 3 * g + ch
                k += 1
    return idx


def kernel(x, y, xstack0, xstack1,
           s0_embed_w, s0_embed_b, s0_ln1_g, s0_ln1_b, s0_ln2_g, s0_ln2_b,
           s0_mlp_w1, s0_mlp_b1, s0_mlp_w2, s0_mlp_b2,
           s1_embed_w, s1_embed_b, s1_ln1_g, s1_ln1_b, s1_ln2_g, s1_ln2_b,
           s1_mlp_w1, s1_mlp_b1, s1_mlp_w2, s1_mlp_b2,
           s2_embed_w, s2_embed_b, s2_ln1_g, s2_ln1_b, s2_ln2_g, s2_ln2_b,
           s2_mlp_w1, s2_mlp_b1, s2_mlp_w2, s2_mlp_b2,
           s3_embed_w, s3_embed_b, s3_ln1_g, s3_ln1_b, s3_ln2_g, s3_ln2_b,
           s3_mlp_w1, s3_mlp_b1, s3_mlp_w2, s3_mlp_b2,
           head_w, head_b):
    stages = [
        dict(embed_w=s0_embed_w, embed_b=s0_embed_b, ln1_g=s0_ln1_g,
             ln1_b=s0_ln1_b, ln2_g=s0_ln2_g, ln2_b=s0_ln2_b,
             mlp_w1=s0_mlp_w1, mlp_b1=s0_mlp_b1, mlp_w2=s0_mlp_w2,
             mlp_b2=s0_mlp_b2),
        dict(embed_w=s1_embed_w, embed_b=s1_embed_b, ln1_g=s1_ln1_g,
             ln1_b=s1_ln1_b, ln2_g=s1_ln2_g, ln2_b=s1_ln2_b,
             mlp_w1=s1_mlp_w1, mlp_b1=s1_mlp_b1, mlp_w2=s1_mlp_w2,
             mlp_b2=s1_mlp_b2),
        dict(embed_w=s2_embed_w, embed_b=s2_embed_b, ln1_g=s2_ln1_g,
             ln1_b=s2_ln1_b, ln2_g=s2_ln2_g, ln2_b=s2_ln2_b,
             mlp_w1=s2_mlp_w1, mlp_b1=s2_mlp_b1, mlp_w2=s2_mlp_w2,
             mlp_b2=s2_mlp_b2),
        dict(embed_w=s3_embed_w, embed_b=s3_embed_b, ln1_g=s3_ln1_g,
             ln1_b=s3_ln1_b, ln2_g=s3_ln2_g, ln2_b=s3_ln2_b,
             mlp_w1=s3_mlp_w1, mlp_b1=s3_mlp_b1, mlp_w2=s3_mlp_w2,
             mlp_b2=s3_mlp_b2),
    ]

    wa = s0_embed_w[jnp.asarray(_s0_perm(0))].T
    wb = s0_embed_w[jnp.asarray(_s0_perm(1))].T
    wc = s0_embed_w[jnp.asarray(_s0_perm(2))].T

    st = stages[0]
    s0_params = (st["embed_b"].reshape(32, 1),
                 st["ln1_g"].reshape(32, 1), st["ln1_b"].reshape(32, 1),
                 st["ln2_g"].reshape(32, 1), st["ln2_b"].reshape(32, 1),
                 st["mlp_w1"].T, st["mlp_b1"].reshape(128, 1),
                 st["mlp_w2"].T, st["mlp_b2"].reshape(32, 1))
    s0_pspecs = [_full((32, 1)), _full((32, 1)), _full((32, 1)),
                 _full((32, 1)), _full((32, 1)), _full((128, 32)),
                 _full((128, 1)), _full((32, 128)), _full((32, 1))]
    out1 = pl.pallas_call(
        _stage0_body,
        out_shape=jax.ShapeDtypeStruct((B, 32, 64, 64), jnp.float32),
        grid=(2, 8),
        in_specs=[
            pl.BlockSpec((1, 3, 256, 256), lambda c, j: (c * 8 + j, 0, 0, 0)),
            pl.BlockSpec((1, 3, 256, 256), lambda c, j: (c * 8 + j, 0, 0, 0)),
            pl.BlockSpec((1, 3, 256, 256), lambda c, j: (c * 8 + j, 0, 0, 0)),
            _full((32, 48)), _full((32, 48)), _full((32, 48)),
        ] + s0_pspecs,
        out_specs=pl.BlockSpec((1, 32, 64, 64), lambda c, j: (c * 8 + j, 0, 0, 0)),
        compiler_params=_compiler_params(),
    )(xstack0, xstack1, y, wa, wb, wc, *s0_params)

    st = stages[1]
    w1T = st["embed_w"].T
    s1_params = (st["embed_b"].reshape(64, 1),
                 st["ln1_g"].reshape(64, 1), st["ln1_b"].reshape(64, 1),
                 st["ln2_g"].reshape(64, 1), st["ln2_b"].reshape(64, 1),
                 st["mlp_w1"].T, st["mlp_b1"].reshape(256, 1),
                 st["mlp_w2"].T, st["mlp_b2"].reshape(64, 1))
    s1_pspecs = [_full((64, 1)), _full((64, 1)), _full((64, 1)),
                 _full((64, 1)), _full((64, 1)), _full((256, 64)),
                 _full((256, 1)), _full((64, 256)), _full((64, 1))]
    out2, p2 = pl.pallas_call(
        _stage1_body,
        out_shape=(jax.ShapeDtypeStruct((B, 64, 32, 32), jnp.float32),
                   jax.ShapeDtypeStruct((B * 256, 256), jnp.bfloat16)),
        grid=(2, 8),
        in_specs=[pl.BlockSpec((1, 32, 64, 64), lambda c, j: (c * 8 + j, 0, 0, 0)),
                  _full((64, 128))] + s1_pspecs,
        out_specs=(pl.BlockSpec((1, 64, 32, 32), lambda c, j: (c * 8 + j, 0, 0, 0)),
                   pl.BlockSpec((256, 256), lambda c, j: (c * 8 + j, 0))),
        compiler_params=_compiler_params(),
    )(out1, w1T, *s1_params)

    oh, ow = OUT_HW
    a_mat, bt_mat = _bilinear_factors(8, 8, oh, ow)
    st2, st3 = stages[2], stages[3]
    out3, out4, rgb = pl.pallas_call(
        _tail_body,
        out_shape=(jax.ShapeDtypeStruct((B, 128, 16, 16), jnp.float32),
                   jax.ShapeDtypeStruct((B, 160, 8, 8), jnp.float32),
                   jax.ShapeDtypeStruct((B, oh, ow), jnp.float32)),
        grid=(2, 1),
        in_specs=[pl.BlockSpec((2048, 256), lambda c, j: (c, 0)),
                  _full((256, 128))] + _stage_param_specs(256, 128)
                 + [_full((512, 160))] + _stage_param_specs(512, 160)
                 + [_full((1, 160)), _full((1, 1)),
                    _full((oh, 8)), _full((8, ow))],
        out_specs=(pl.BlockSpec((8, 128, 16, 16), lambda c, j: (c, 0, 0, 0)),
                   pl.BlockSpec((8, 160, 8, 8), lambda c, j: (c, 0, 0, 0)),
                   pl.BlockSpec((8, oh, ow), lambda c, j: (c, 0, 0))),
        compiler_params=_compiler_params(),
    )(p2, st2["embed_w"], *_stage_params(st2, 128), st3["embed_w"],
      *_stage_params(st3, 160), head_w.reshape(1, 160),
      head_b.reshape(1, 1), a_mat, bt_mat)

    return rgb.reshape(B, 1, oh, ow), out1, out2, out3, out4
